# baseline (speedup 1.0000x reference)
import functools

import jax
import jax.numpy as jnp
from jax import lax
from jax.experimental import pallas as pl
from jax.experimental.pallas import tpu as pltpu
from jax.experimental.pallas import tpu_sc as plsc

EMB = 64
HID = 256
LAB = 16
B = 4096
L = 200

NC = 2
NS = 16
NW = NC * NS
EPW = B // NW
CH = 104
CHR = 100
NCHUNK = 2 * EPW


def _pool_body(idx_hbm, table_hbm, out_hbm, idx_v, rows0, rows1, pooled_v,
               sem0, sem1):
    c = lax.axis_index("c")
    s = lax.axis_index("s")
    wid = c * NS + s

    pltpu.sync_copy(idx_hbm.at[wid], idx_v)

    def accumulate(rows, accs):
        def body(t, a):
            return (a[0] + rows[t, pl.ds(0, 16)],
                    a[1] + rows[t, pl.ds(16, 16)],
                    a[2] + rows[t, pl.ds(32, 16)],
                    a[3] + rows[t, pl.ds(48, 16)])
        return lax.fori_loop(0, CHR, body, accs)

    def example(e, carry):
        pltpu.async_copy(table_hbm.at[idx_v.at[2 * e]], rows0, sem0).wait()
        pltpu.async_copy(table_hbm.at[idx_v.at[2 * e + 1]], rows1, sem1).wait()
        z = jnp.zeros((16,), jnp.float32)
        accs = accumulate(rows0, (z, z, z, z))
        accs = accumulate(rows1, accs)
        for j in range(4):
            pooled_v[e, pl.ds(16 * j, 16)] = accs[j]
        return carry

    lax.fori_loop(0, EPW, example, 0)
    pltpu.sync_copy(pooled_v, out_hbm.at[pl.ds(wid * EPW, EPW)])


@jax.jit
def _pooled_sums(idx_padded, table):
    mesh = plsc.VectorSubcoreMesh(core_axis_name="c", subcore_axis_name="s")
    f = pl.kernel(
        _pool_body,
        out_type=jax.ShapeDtypeStruct((B, EMB), jnp.float32),
        mesh=mesh,
        scratch_types=[
            pltpu.VMEM((NCHUNK, CH), jnp.int32),
            pltpu.VMEM((CH, EMB), jnp.float32),
            pltpu.VMEM((CH, EMB), jnp.float32),
            pltpu.VMEM((EPW, EMB), jnp.float32),
            pltpu.SemaphoreType.DMA,
            pltpu.SemaphoreType.DMA,
        ],
        compiler_params=pltpu.CompilerParams(use_tc_tiling_on_sc=False),
    )
    return f(idx_padded, table)


def _mlp_body(x_ref, w1_ref, b1_ref, w2_ref, b2_ref, o_ref):
    x = x_ref[...] * (1.0 / L)
    h = jnp.dot(x, w1_ref[...], preferred_element_type=jnp.float32)
    h = jnp.maximum(h + b1_ref[...], 0.0)
    o = jnp.dot(h, w2_ref[...], preferred_element_type=jnp.float32)
    o_ref[...] = o + b2_ref[...]


@jax.jit
def _mlp(pooled, W1, b1, W2, b2):
    blk = 512
    grid = B // blk
    return pl.pallas_call(
        _mlp_body,
        out_shape=jax.ShapeDtypeStruct((B, LAB), jnp.float32),
        grid=(grid,),
        in_specs=[
            pl.BlockSpec((blk, EMB), lambda i: (i, 0)),
            pl.BlockSpec((EMB, HID), lambda i: (0, 0)),
            pl.BlockSpec((1, HID), lambda i: (0, 0)),
            pl.BlockSpec((HID, LAB), lambda i: (0, 0)),
            pl.BlockSpec((1, LAB), lambda i: (0, 0)),
        ],
        out_specs=pl.BlockSpec((blk, LAB), lambda i: (i, 0)),
    )(pooled, W1, b1.reshape(1, HID), W2, b2.reshape(1, LAB))


def kernel(input_id, mask, table, W1, b1, W2, b2):
    del mask
    idx = input_id.astype(jnp.int32).reshape(B * 2, CHR)
    idx = jnp.pad(idx, ((0, 0), (0, CH - CHR)))
    idx = idx.reshape(NW, NCHUNK, CH)
    pooled = _pooled_sums(idx, table)
    return _mlp(pooled, W1, b1, W2, b2)
